# baseline (speedup 1.0000x reference)
import functools
import math

import jax
import jax.numpy as jnp
from jax import lax
from jax.experimental import pallas as pl
from jax.experimental.pallas import tpu as pltpu
from jax.experimental.pallas import tpu_sc as plsc

NC = 2
NS = 16
L = 16
NW = NC * NS

B = 4
T = 8192
D = 768
SCALE = math.sqrt(float(D))

TOK_PER_W = (B * T) // NW
CH = 32
NCH = TOK_PER_W // CH
VPR = D // L


def _emb_kernel(ids_hbm, tok_hbm, pos_hbm, out_hbm,
                idx_v, tok_buf, pos_buf, sem):
    wid = lax.axis_index("s") * NC + lax.axis_index("c")
    base = wid * TOK_PER_W
    tbase = lax.rem(base, T)

    pltpu.sync_copy(ids_hbm.at[pl.ds(base, TOK_PER_W)], idx_v)

    @pl.loop(0, NCH)
    def _chunk(j):
        off = j * CH
        gcp = pltpu.async_copy(
            tok_hbm.at[idx_v.at[pl.ds(off, CH)]], tok_buf, sem)
        pltpu.sync_copy(pos_hbm.at[pl.ds(tbase + off, CH)], pos_buf)
        gcp.wait()

        @pl.loop(0, CH)
        def _row(r):
            for k in range(VPR):
                sl = pl.ds(k * L, L)
                tok_buf[r, sl] = tok_buf[r, sl] * SCALE + pos_buf[r, sl]

        pltpu.sync_copy(tok_buf, out_hbm.at[pl.ds(base + off, CH)])


@jax.jit
def _emb_call(ids_flat, token_embed, pos_embed):
    mesh = plsc.VectorSubcoreMesh(core_axis_name="c", subcore_axis_name="s")
    fn = pl.kernel(
        _emb_kernel,
        out_type=jax.ShapeDtypeStruct((B * T, D), jnp.float32),
        mesh=mesh,
        scratch_types=[
            pltpu.VMEM((TOK_PER_W,), jnp.int32),
            pltpu.VMEM((CH, D), jnp.float32),
            pltpu.VMEM((CH, D), jnp.float32),
            pltpu.SemaphoreType.DMA,
        ],
    )
    return fn(ids_flat, token_embed, pos_embed)


def kernel(input_ids, token_embed, pos_embed):
    ids_flat = input_ids.astype(jnp.int32).reshape(B * T)
    out = _emb_call(ids_flat, token_embed, pos_embed)
    return out.reshape(B, T, D)


# speedup vs baseline: 1.6329x; 1.6329x over previous
import math

import jax
import jax.numpy as jnp
from jax import lax
from jax.experimental import pallas as pl
from jax.experimental.pallas import tpu as pltpu
from jax.experimental.pallas import tpu_sc as plsc

NC = 2
NS = 16
L = 16
NW = NC * NS

B = 4
T = 8192
D = 768
SCALE = math.sqrt(float(D))

TPW = T // NW
CH = 32
NTC = TPW // CH
VPR = D // L


def _emb_kernel(ids_hbm, tok_hbm, pos_hbm, out_hbm,
                idx_v, tok0, tok1, pos0, pos1,
                gs0, gs1, ps0, ps1, os0, os1):
    wid = lax.axis_index("s") * NC + lax.axis_index("c")
    t0 = wid * TPW

    for b in range(B):
        pltpu.sync_copy(ids_hbm.at[pl.ds(b * T + t0, TPW)],
                        idx_v.at[pl.ds(b * TPW, TPW)])

    toks = (tok0, tok1)
    poss = (pos0, pos1)
    gsems = (gs0, gs1)
    psems = (ps0, ps1)
    osems = (os0, os1)

    pltpu.async_copy(pos_hbm.at[pl.ds(t0, CH)], pos0, ps0)
    pltpu.async_copy(tok_hbm.at[idx_v.at[pl.ds(0, CH)]], tok0, gs0)

    @pl.loop(0, NTC, step=2)
    def _tc2(tc0):
        for tcu in range(2):
            tc = tc0 + tcu
            posbuf, psem = poss[tcu], psems[tcu]
            nposbuf, npsem = poss[1 - tcu], psems[1 - tcu]
            for b in range(B):
                u = b % 2
                v = 1 - u
                tbuf = toks[u]

                if b < B - 1:
                    noff = (b + 1) * TPW + tc * CH
                    if b == 0:
                        @pl.when(tc > 0)
                        def _():
                            pltpu.make_async_copy(
                                toks[v], out_hbm.at[pl.ds(0, CH)],
                                osems[v]).wait()
                    else:
                        pltpu.make_async_copy(
                            toks[v], out_hbm.at[pl.ds(0, CH)],
                            osems[v]).wait()
                    pltpu.async_copy(
                        tok_hbm.at[idx_v.at[pl.ds(noff, CH)]],
                        toks[v], gsems[v])
                else:
                    @pl.when(tc < NTC - 1)
                    def _():
                        noff = (tc + 1) * CH
                        pltpu.make_async_copy(
                            toks[v], out_hbm.at[pl.ds(0, CH)],
                            osems[v]).wait()
                        pltpu.async_copy(
                            tok_hbm.at[idx_v.at[pl.ds(noff, CH)]],
                            toks[v], gsems[v])

                if b == 1:
                    @pl.when(tc < NTC - 1)
                    def _():
                        pltpu.async_copy(
                            pos_hbm.at[pl.ds(t0 + (tc + 1) * CH, CH)],
                            nposbuf, npsem)

                pltpu.make_async_copy(
                    tok_hbm.at[pl.ds(0, CH)], tbuf, gsems[u]).wait()
                if b == 0:
                    pltpu.make_async_copy(
                        pos_hbm.at[pl.ds(0, CH)], posbuf, psem).wait()

                @pl.loop(0, CH)
                def _row(r):
                    for k in range(VPR):
                        sl = pl.ds(k * L, L)
                        tbuf[r, sl] = tbuf[r, sl] * SCALE + posbuf[r, sl]

                pltpu.async_copy(
                    tbuf, out_hbm.at[pl.ds(b * T + t0 + tc * CH, CH)],
                    osems[u])

    pltpu.make_async_copy(tok0, out_hbm.at[pl.ds(0, CH)], os0).wait()
    pltpu.make_async_copy(tok1, out_hbm.at[pl.ds(0, CH)], os1).wait()


@jax.jit
def _emb_call(ids_flat, token_embed, pos_embed):
    mesh = plsc.VectorSubcoreMesh(core_axis_name="c", subcore_axis_name="s")
    fn = pl.kernel(
        _emb_kernel,
        out_type=jax.ShapeDtypeStruct((B * T, D), jnp.float32),
        mesh=mesh,
        scratch_types=[
            pltpu.VMEM((B * TPW,), jnp.int32),
            pltpu.VMEM((CH, D), jnp.float32),
            pltpu.VMEM((CH, D), jnp.float32),
            pltpu.VMEM((CH, D), jnp.float32),
            pltpu.VMEM((CH, D), jnp.float32),
            pltpu.SemaphoreType.DMA,
            pltpu.SemaphoreType.DMA,
            pltpu.SemaphoreType.DMA,
            pltpu.SemaphoreType.DMA,
            pltpu.SemaphoreType.DMA,
            pltpu.SemaphoreType.DMA,
        ],
    )
    return fn(ids_flat, token_embed, pos_embed)


def kernel(input_ids, token_embed, pos_embed):
    ids_flat = input_ids.astype(jnp.int32).reshape(B * T)
    out = _emb_call(ids_flat, token_embed, pos_embed)
    return out.reshape(B, T, D)
